# baseline (device time: 136786 ns/iter reference)
import numpy as np

import jax
import jax.numpy as jnp
from jax import lax
from jax.experimental import pallas as pl
from jax.experimental.pallas import tpu as pltpu

N_DEV = 16
N_PLANES = 4
PER_PLANE = 4
BM = 512
CHUNK = 2048

_ORDER_TABLE = np.zeros((N_PLANES, N_DEV), np.int32)
for _p in range(N_PLANES):
    _ORDER_TABLE[_p] = sorted(
        range(N_DEV), key=lambda j: (abs(j // PER_PLANE - _p), j)
    )


def kernel(x, w_mat):
    m, k_per = x.shape
    k_tot, n = w_mat.shape
    n_chunks = m // CHUNK

    def body(order_ref, x_hbm, w_ref, out_ref, xf_ref, xb_ref, comm_ref,
             xsems, send_sems, recv_sems):
        k = pl.program_id(0)
        me = lax.axis_index("i")
        my_plane = me // PER_PLANE
        my_col = me % PER_PLANE
        j = order_ref[k]

        def send_block(d):
            rdma = pltpu.make_async_remote_copy(
                src_ref=xb_ref.at[pl.ds(d * BM, BM), :],
                dst_ref=comm_ref.at[me],
                send_sem=send_sems.at[d],
                recv_sem=recv_sems.at[me],
                device_id=(d,),
                device_id_type=pl.DeviceIdType.MESH,
            )
            rdma.start()

        @pl.when(k == 0)
        def _first():
            out_ref[...] = jnp.zeros_like(out_ref)
            pltpu.make_async_copy(
                x_hbm.at[pl.ds(0, CHUNK), :], xf_ref.at[0], xsems.at[0]
            ).start()
            for c in range(n_chunks):
                if c + 1 < n_chunks:
                    pltpu.make_async_copy(
                        x_hbm.at[pl.ds((c + 1) * CHUNK, CHUNK), :],
                        xf_ref.at[(c + 1) % 2],
                        xsems.at[(c + 1) % 2],
                    ).start()
                pltpu.make_async_copy(
                    x_hbm.at[pl.ds(c * CHUNK, CHUNK), :],
                    xf_ref.at[c % 2],
                    xsems.at[c % 2],
                ).wait()
                xb_ref[pl.ds(c * CHUNK, CHUNK), :] = (
                    xf_ref[c % 2].astype(jnp.bfloat16)
                )
            for d in range(N_DEV):
                @pl.when(jnp.logical_and(my_plane != d // PER_PLANE,
                                         my_col == d % PER_PLANE))
                def _s1(d=d):
                    send_block(d)
            for d in range(N_DEV):
                @pl.when(jnp.logical_and(my_plane != d // PER_PLANE,
                                         my_col != d % PER_PLANE))
                def _s2(d=d):
                    send_block(d)
            for d in range(N_DEV):
                @pl.when(jnp.logical_and(my_plane == d // PER_PLANE,
                                         me != d))
                def _s3(d=d):
                    send_block(d)

        @pl.when(j != me)
        def _wait():
            recv = pltpu.make_async_remote_copy(
                src_ref=comm_ref.at[j],
                dst_ref=comm_ref.at[j],
                send_sem=send_sems.at[j],
                recv_sem=recv_sems.at[j],
                device_id=(me,),
                device_id_type=pl.DeviceIdType.MESH,
            )
            recv.wait_recv()

        a_own = xb_ref[pl.ds(me * BM, BM), :]
        a = jnp.where(j == me, a_own, comm_ref[j])
        wb = w_ref[...].astype(jnp.bfloat16)
        out_ref[...] += jnp.dot(a, wb, preferred_element_type=jnp.float32)

        @pl.when(k == N_DEV - 1)
        def _fin():
            for d in range(N_DEV):
                @pl.when(d != me)
                def _wait_send(d=d):
                    s = pltpu.make_async_remote_copy(
                        src_ref=xb_ref.at[pl.ds(d * BM, BM), :],
                        dst_ref=comm_ref.at[me],
                        send_sem=send_sems.at[d],
                        recv_sem=recv_sems.at[me],
                        device_id=(d,),
                        device_id_type=pl.DeviceIdType.MESH,
                    )
                    s.wait_send()
            y = out_ref[...]
            out_ref[...] = y * (1.0 / (1.0 + jnp.exp(-y)))

    me_out = lax.axis_index("i")
    order = jnp.asarray(_ORDER_TABLE)[me_out // PER_PLANE]

    grid_spec = pltpu.PrefetchScalarGridSpec(
        num_scalar_prefetch=1,
        grid=(N_DEV,),
        in_specs=[
            pl.BlockSpec(memory_space=pl.ANY),
            pl.BlockSpec((BM, n), lambda k, order: (order[k], 0)),
        ],
        out_specs=pl.BlockSpec((BM, n), lambda k, order: (0, 0)),
        scratch_shapes=[
            pltpu.VMEM((2, CHUNK, k_per), jnp.float32),
            pltpu.VMEM((m, k_per), jnp.bfloat16),
            pltpu.VMEM((N_DEV, BM, k_per), jnp.bfloat16),
            pltpu.SemaphoreType.DMA((2,)),
            pltpu.SemaphoreType.DMA((N_DEV,)),
            pltpu.SemaphoreType.DMA((N_DEV,)),
        ],
    )

    return pl.pallas_call(
        body,
        grid_spec=grid_spec,
        out_shape=jax.ShapeDtypeStruct((BM, n), jnp.float32),
        compiler_params=pltpu.CompilerParams(
            dimension_semantics=("arbitrary",),
            vmem_limit_bytes=56 * 1024 * 1024,
        ),
    )(order, x, w_mat)


# device time: 126036 ns/iter; 1.0853x vs baseline; 1.0853x over previous
import os

import numpy as np

import jax
import jax.numpy as jnp
from jax import lax
from jax.experimental import pallas as pl
from jax.experimental.pallas import tpu as pltpu

_KV = os.environ.get("KV", "full")
DO_COMM = _KV in ("full", "comm")
DO_COMPUTE = _KV in ("full", "compute")

N_DEV = 16
N_PLANES = 4
PER_PLANE = 4
BM = 512
CHUNK = 2048

_ORDER_TABLE = np.zeros((N_PLANES, N_DEV), np.int32)
for _p in range(N_PLANES):
    _ORDER_TABLE[_p] = sorted(
        range(N_DEV), key=lambda j: (abs(j // PER_PLANE - _p), j)
    )


def kernel(x, w_mat):
    m, k_per = x.shape
    k_tot, n = w_mat.shape
    n_chunks = m // CHUNK

    def body(order_ref, x_hbm, w_ref, out_ref, xf_ref, xb_ref, comm_ref,
             xsems, send_sems, recv_sems):
        k = pl.program_id(0)
        me = lax.axis_index("i")
        my_plane = me // PER_PLANE
        my_col = me % PER_PLANE
        j = order_ref[k]

        def send_block(d):
            rdma = pltpu.make_async_remote_copy(
                src_ref=xb_ref.at[pl.ds(d * BM, BM), :],
                dst_ref=comm_ref.at[me],
                send_sem=send_sems.at[d],
                recv_sem=recv_sems.at[me],
                device_id=(d,),
                device_id_type=pl.DeviceIdType.MESH,
            )
            rdma.start()

        @pl.when(k == 0)
        def _first():
            out_ref[...] = jnp.zeros_like(out_ref)
            pltpu.make_async_copy(
                x_hbm.at[pl.ds(0, CHUNK), :], xf_ref.at[0], xsems.at[0]
            ).start()
            for c in range(n_chunks):
                if c + 1 < n_chunks:
                    pltpu.make_async_copy(
                        x_hbm.at[pl.ds((c + 1) * CHUNK, CHUNK), :],
                        xf_ref.at[(c + 1) % 2],
                        xsems.at[(c + 1) % 2],
                    ).start()
                pltpu.make_async_copy(
                    x_hbm.at[pl.ds(c * CHUNK, CHUNK), :],
                    xf_ref.at[c % 2],
                    xsems.at[c % 2],
                ).wait()
                xb_ref[pl.ds(c * CHUNK, CHUNK), :] = (
                    xf_ref[c % 2].astype(jnp.bfloat16)
                )
            if not DO_COMM:
                return
            for d in range(N_DEV):
                @pl.when(jnp.logical_and(my_plane != d // PER_PLANE,
                                         my_col == d % PER_PLANE))
                def _s1(d=d):
                    send_block(d)
            for d in range(N_DEV):
                @pl.when(jnp.logical_and(my_plane != d // PER_PLANE,
                                         my_col != d % PER_PLANE))
                def _s2(d=d):
                    send_block(d)
            for d in range(N_DEV):
                @pl.when(jnp.logical_and(my_plane == d // PER_PLANE,
                                         me != d))
                def _s3(d=d):
                    send_block(d)

        @pl.when(jnp.logical_and(j != me, DO_COMM))
        def _wait():
            recv = pltpu.make_async_remote_copy(
                src_ref=comm_ref.at[j],
                dst_ref=comm_ref.at[j],
                send_sem=send_sems.at[j],
                recv_sem=recv_sems.at[j],
                device_id=(me,),
                device_id_type=pl.DeviceIdType.MESH,
            )
            recv.wait_recv()

        if DO_COMPUTE:
            a_own = xb_ref[pl.ds(me * BM, BM), :]
            a = jnp.where(j == me, a_own, comm_ref[j])
            wb = w_ref[...].astype(jnp.bfloat16)
            out_ref[...] += jnp.dot(a, wb, preferred_element_type=jnp.float32)

        @pl.when(jnp.logical_and(k == N_DEV - 1, DO_COMM))
        def _fin():
            for d in range(N_DEV):
                @pl.when(d != me)
                def _wait_send(d=d):
                    s = pltpu.make_async_remote_copy(
                        src_ref=xb_ref.at[pl.ds(d * BM, BM), :],
                        dst_ref=comm_ref.at[me],
                        send_sem=send_sems.at[d],
                        recv_sem=recv_sems.at[me],
                        device_id=(d,),
                        device_id_type=pl.DeviceIdType.MESH,
                    )
                    s.wait_send()

        if DO_COMPUTE:
            @pl.when(k == N_DEV - 1)
            def _silu():
                y = out_ref[...]
                out_ref[...] = y * (1.0 / (1.0 + jnp.exp(-y)))

    me_out = lax.axis_index("i")
    order = jnp.asarray(_ORDER_TABLE)[me_out // PER_PLANE]

    grid_spec = pltpu.PrefetchScalarGridSpec(
        num_scalar_prefetch=1,
        grid=(N_DEV,),
        in_specs=[
            pl.BlockSpec(memory_space=pl.ANY),
            pl.BlockSpec((BM, n), lambda k, order: (order[k], 0)),
        ],
        out_specs=pl.BlockSpec((BM, n), lambda k, order: (0, 0)),
        scratch_shapes=[
            pltpu.VMEM((2, CHUNK, k_per), jnp.float32),
            pltpu.VMEM((m, k_per), jnp.bfloat16),
            pltpu.VMEM((N_DEV, BM, k_per), jnp.bfloat16),
            pltpu.SemaphoreType.DMA((2,)),
            pltpu.SemaphoreType.DMA((N_DEV,)),
            pltpu.SemaphoreType.DMA((N_DEV,)),
        ],
    )

    return pl.pallas_call(
        body,
        grid_spec=grid_spec,
        out_shape=jax.ShapeDtypeStruct((BM, n), jnp.float32),
        compiler_params=pltpu.CompilerParams(
            dimension_semantics=("arbitrary",),
            vmem_limit_bytes=56 * 1024 * 1024,
        ),
    )(order, x, w_mat)
